# baseline (speedup 1.0000x reference)
import functools

import jax
import jax.numpy as jnp
from jax.experimental import pallas as pl
from jax.experimental.pallas import tpu as pltpu

_B, _NQ, _C = 64, 5000, 80
_K = 300
_LANES = 128
_BLK_ROWS = 32
_SLOTS = 512
_BIG = 1 << 30


def _body(nrows, nblk, nq, c, k, logits_ref, boxes_ref, scale_ref,
          scores_out, labels_out, boxsel_out, si_ref, y_ref, bm_ref):
    s = jax.nn.sigmoid(logits_ref[0])
    si = jax.lax.bitcast_convert_type(s, jnp.int32)
    si_ref[...] = si

    def thresh_body(_, carry):
        lo, hi = carry
        mid = (lo + hi) // 2
        cnt = jnp.sum((si_ref[...] > mid).astype(jnp.int32))
        pred = cnt < k
        return (jnp.where(pred, lo, mid), jnp.where(pred, mid, hi))

    lo0 = jnp.int32(-1)
    hi0 = jnp.int32(0x3F800000)
    _, tbits = jax.lax.fori_loop(0, 31, thresh_body, (lo0, hi0))

    m = jnp.sum((si_ref[...] > tbits).astype(jnp.int32))
    ties_needed = k - m

    fi = (jax.lax.broadcasted_iota(jnp.int32, (nrows, _LANES), 0) * _LANES
          + jax.lax.broadcasted_iota(jnp.int32, (nrows, _LANES), 1))
    y_ref[...] = jnp.where(si_ref[...] == tbits, fi, jnp.int32(_BIG))

    def tie_body(_, carry):
        lo, hi = carry
        mid = (lo + hi) // 2
        cnt = jnp.sum((y_ref[...] < mid).astype(jnp.int32))
        pred = cnt >= ties_needed
        return (jnp.where(pred, lo, mid), jnp.where(pred, mid, hi))

    _, istar = jax.lax.fori_loop(
        0, 20, tie_body, (jnp.int32(-1), jnp.int32(nrows * _LANES)))

    sel = (si_ref[...] > tbits) | ((si_ref[...] == tbits) & (fi < istar))
    y_ref[...] = jnp.where(sel, si_ref[...], jnp.int32(-1))

    yb = y_ref[...].reshape(nblk, _BLK_ROWS, _LANES)
    bm0 = jnp.max(yb, axis=(1, 2))
    bmrow = jnp.concatenate(
        [bm0, jnp.full((_LANES - nblk,), -1, jnp.int32)]).reshape(1, _LANES)
    bm_ref[0:1, :] = bmrow

    lane_iota = jax.lax.broadcasted_iota(jnp.int32, (1, _LANES), 1)
    slot_iota = jax.lax.broadcasted_iota(jnp.int32, (1, _SLOTS), 1)
    fib = (jax.lax.broadcasted_iota(jnp.int32, (_BLK_ROWS, _LANES), 0) * _LANES
           + jax.lax.broadcasted_iota(jnp.int32, (_BLK_ROWS, _LANES), 1))

    def extract_body(cslot, carry):
        sv, fv = carry
        bmr = bm_ref[0:1, :]
        v = jnp.max(bmr)
        kstar = jnp.min(jnp.where(bmr == v, lane_iota, jnp.int32(_LANES)))
        blk = y_ref[pl.ds(kstar * _BLK_ROWS, _BLK_ROWS), :]
        p = jnp.min(jnp.where(blk == v, fib, jnp.int32(_BIG)))
        fstar = kstar * (_BLK_ROWS * _LANES) + p
        oh = slot_iota == cslot
        sv = sv + jnp.where(oh, v, 0)
        fv = fv + jnp.where(oh, fstar, 0)
        blk2 = jnp.where(fib == p, jnp.int32(-1), blk)
        y_ref[pl.ds(kstar * _BLK_ROWS, _BLK_ROWS), :] = blk2
        newmax = jnp.max(blk2)
        bm_ref[0:1, :] = jnp.where(lane_iota == kstar, newmax, bmr)
        return (sv, fv)

    sv0 = jnp.zeros((1, _SLOTS), jnp.int32)
    sv, fv = jax.lax.fori_loop(0, k, extract_body, (sv0, sv0))

    scores_out[0, 0:1, :] = jax.lax.bitcast_convert_type(sv, jnp.float32)
    q = fv // c
    labels_out[0, 0:1, :] = fv - q * c

    br = boxes_ref[0]
    w_sc = scale_ref[0, 0, 0]
    h_sc = scale_ref[0, 1, 0]
    cx, cy, bw, bh = br[0:1], br[1:2], br[2:3], br[3:4]
    x1 = (cx - 0.5 * bw) * w_sc
    y1 = (cy - 0.5 * bh) * h_sc
    x2 = (cx + 0.5 * bw) * w_sc
    y2 = (cy + 0.5 * bh) * h_sc
    xyxy = jnp.concatenate([x1, y1, x2, y2, jnp.zeros((4, nq), jnp.float32)])
    q_iota = jax.lax.broadcasted_iota(jnp.int32, (nq, _SLOTS), 0)
    oht = (q_iota == q).astype(jnp.float32)
    boxsel_out[0] = jax.lax.dot_general(
        xyxy, oht, (((1,), (0,)), ((), ())),
        preferred_element_type=jnp.float32)


def _run(pred_logits, pred_boxes, orig_target_sizes, k):
    b, nq, c = pred_logits.shape
    flat = pred_logits.reshape(b, (nq * c) // _LANES, _LANES)
    nrows = -(-flat.shape[1] // _BLK_ROWS) * _BLK_ROWS
    nblk = nrows // _BLK_ROWS
    flat = jnp.pad(flat, ((0, 0), (0, nrows - flat.shape[1]), (0, 0)),
                   constant_values=-1e30)
    boxes_t = jnp.pad(jnp.transpose(pred_boxes, (0, 2, 1)),
                      ((0, 0), (0, 4), (0, 0)))
    sizes_f = orig_target_sizes.astype(jnp.float32)
    scale = jnp.broadcast_to(
        jnp.pad(sizes_f, ((0, 0), (0, 6)))[:, :, None], (b, 8, _LANES))

    body = functools.partial(_body, nrows, nblk, nq, c, k)
    scores, labels, boxsel = pl.pallas_call(
        body,
        grid=(b,),
        in_specs=[
            pl.BlockSpec((1, nrows, _LANES), lambda i: (i, 0, 0)),
            pl.BlockSpec((1, 8, nq), lambda i: (i, 0, 0)),
            pl.BlockSpec((1, 8, _LANES), lambda i: (i, 0, 0)),
        ],
        out_specs=[
            pl.BlockSpec((1, 8, _SLOTS), lambda i: (i, 0, 0)),
            pl.BlockSpec((1, 8, _SLOTS), lambda i: (i, 0, 0)),
            pl.BlockSpec((1, 8, _SLOTS), lambda i: (i, 0, 0)),
        ],
        out_shape=[
            jax.ShapeDtypeStruct((b, 8, _SLOTS), jnp.float32),
            jax.ShapeDtypeStruct((b, 8, _SLOTS), jnp.int32),
            jax.ShapeDtypeStruct((b, 8, _SLOTS), jnp.float32),
        ],
        scratch_shapes=[
            pltpu.VMEM((nrows, _LANES), jnp.int32),
            pltpu.VMEM((nrows, _LANES), jnp.int32),
            pltpu.VMEM((8, _LANES), jnp.int32),
        ],
        compiler_params=pltpu.CompilerParams(
            dimension_semantics=("parallel",)),
    )(flat, boxes_t, scale)

    top_scores = scores[:, 0, :k]
    out_labels = labels[:, 0, :k]
    out_boxes = jnp.transpose(boxsel[:, :4, :k], (0, 2, 1))
    return (out_labels, out_boxes, top_scores)


def kernel(pred_logits, pred_boxes, orig_target_sizes, threshold):
    return _run(pred_logits, pred_boxes, orig_target_sizes, _K)


# speedup vs baseline: 1.0328x; 1.0328x over previous
import functools

import jax
import jax.numpy as jnp
from jax.experimental import pallas as pl
from jax.experimental.pallas import tpu as pltpu

_B, _NQ, _C = 64, 5000, 80
_K = 300
_LANES = 128
_BLK_ROWS = 32
_SLOTS = 512
_BIG = 1 << 30


def _body(nrows, nblk, nq, c, k, logits_ref, boxes_ref, scale_ref,
          scores_out, labels_out, boxsel_out, si_ref, h16_ref, y_ref, bm_ref):
    s = jax.nn.sigmoid(logits_ref[0])
    si = jax.lax.bitcast_convert_type(s, jnp.int32)
    si_ref[...] = si
    h16_ref[...] = (si >> 16).astype(jnp.int16)

    def count16(mid):
        cmp = (h16_ref[...] > mid.astype(jnp.int16)).astype(jnp.int16)
        return jnp.sum(jnp.sum(cmp, axis=0).astype(jnp.int32))

    def bis_body(cntf, kk):
        def body(_, carry):
            lo, hi = carry
            mid = (lo + hi) // 2
            pred = cntf(mid) < kk
            return (jnp.where(pred, lo, mid), jnp.where(pred, mid, hi))
        return body

    _, t16 = jax.lax.fori_loop(
        0, 15, bis_body(count16, k), (jnp.int32(-1), jnp.int32(0x3F80)))
    m_hi = count16(t16)
    kb = k - m_hi

    low = jnp.where(si >> 16 == t16, (si & 0xFFFF) - 32768, -32768)
    h16_ref[...] = low.astype(jnp.int16)
    _, tlo = jax.lax.fori_loop(
        0, 17, bis_body(count16, kb), (jnp.int32(-32769), jnp.int32(32767)))
    tbits = (t16 << 16) + (tlo + 32768)

    m = jnp.sum((si_ref[...] > tbits).astype(jnp.int32))
    eq_total = jnp.sum((si_ref[...] == tbits).astype(jnp.int32))
    ties_needed = k - m

    fi = (jax.lax.broadcasted_iota(jnp.int32, (nrows, _LANES), 0) * _LANES
          + jax.lax.broadcasted_iota(jnp.int32, (nrows, _LANES), 1))

    def tie_bisect():
        def tie_body(_, carry):
            lo, hi = carry
            mid = (lo + hi) // 2
            eqfi = jnp.where(si_ref[...] == tbits, fi, jnp.int32(_BIG))
            cnt = jnp.sum((eqfi < mid).astype(jnp.int32))
            pred = cnt >= ties_needed
            return (jnp.where(pred, lo, mid), jnp.where(pred, mid, hi))
        return jax.lax.fori_loop(
            0, 20, tie_body, (jnp.int32(-1), jnp.int32(nrows * _LANES)))[1]

    istar = jax.lax.cond(eq_total == ties_needed,
                         lambda: jnp.int32(_BIG), tie_bisect)

    sel = (si_ref[...] > tbits) | ((si_ref[...] == tbits) & (fi < istar))
    y_ref[...] = jnp.where(sel, si_ref[...], jnp.int32(-1))

    yb = y_ref[...].reshape(nblk, _BLK_ROWS, _LANES)
    bm0 = jnp.max(yb, axis=(1, 2))
    bmrow = jnp.concatenate(
        [bm0, jnp.full((_LANES - nblk,), -1, jnp.int32)]).reshape(1, _LANES)
    bm_ref[0:1, :] = bmrow

    lane_iota = jax.lax.broadcasted_iota(jnp.int32, (1, _LANES), 1)
    slot_iota = jax.lax.broadcasted_iota(jnp.int32, (1, _SLOTS), 1)
    fib = (jax.lax.broadcasted_iota(jnp.int32, (_BLK_ROWS, _LANES), 0) * _LANES
           + jax.lax.broadcasted_iota(jnp.int32, (_BLK_ROWS, _LANES), 1))

    def extract_body(cslot, carry):
        sv, fv = carry
        bmr = bm_ref[0:1, :]
        v = jnp.max(bmr)
        kstar = jnp.min(jnp.where(bmr == v, lane_iota, jnp.int32(_LANES)))
        blk = y_ref[pl.ds(kstar * _BLK_ROWS, _BLK_ROWS), :]
        p = jnp.min(jnp.where(blk == v, fib, jnp.int32(_BIG)))
        fstar = kstar * (_BLK_ROWS * _LANES) + p
        oh = slot_iota == cslot
        sv = sv + jnp.where(oh, v, 0)
        fv = fv + jnp.where(oh, fstar, 0)
        blk2 = jnp.where(fib == p, jnp.int32(-1), blk)
        y_ref[pl.ds(kstar * _BLK_ROWS, _BLK_ROWS), :] = blk2
        newmax = jnp.max(blk2)
        bm_ref[0:1, :] = jnp.where(lane_iota == kstar, newmax, bmr)
        return (sv, fv)

    sv0 = jnp.zeros((1, _SLOTS), jnp.int32)
    sv, fv = jax.lax.fori_loop(0, k, extract_body, (sv0, sv0))

    scores_out[0, 0:1, :] = jax.lax.bitcast_convert_type(sv, jnp.float32)
    q = fv // c
    labels_out[0, 0:1, :] = fv - q * c

    br = boxes_ref[0]
    w_sc = scale_ref[0, 0, 0]
    h_sc = scale_ref[0, 1, 0]
    cx, cy, bw, bh = br[0:1], br[1:2], br[2:3], br[3:4]
    x1 = (cx - 0.5 * bw) * w_sc
    y1 = (cy - 0.5 * bh) * h_sc
    x2 = (cx + 0.5 * bw) * w_sc
    y2 = (cy + 0.5 * bh) * h_sc
    xyxy = jnp.concatenate([x1, y1, x2, y2, jnp.zeros((4, nq), jnp.float32)])
    q_iota = jax.lax.broadcasted_iota(jnp.int32, (nq, _SLOTS), 0)
    oht = (q_iota == q).astype(jnp.float32)
    boxsel_out[0] = jax.lax.dot_general(
        xyxy, oht, (((1,), (0,)), ((), ())),
        preferred_element_type=jnp.float32)


def _run(pred_logits, pred_boxes, orig_target_sizes, k):
    b, nq, c = pred_logits.shape
    flat = pred_logits.reshape(b, (nq * c) // _LANES, _LANES)
    nrows = -(-flat.shape[1] // _BLK_ROWS) * _BLK_ROWS
    nblk = nrows // _BLK_ROWS
    flat = jnp.pad(flat, ((0, 0), (0, nrows - flat.shape[1]), (0, 0)),
                   constant_values=-1e30)
    boxes_t = jnp.pad(jnp.transpose(pred_boxes, (0, 2, 1)),
                      ((0, 0), (0, 4), (0, 0)))
    sizes_f = orig_target_sizes.astype(jnp.float32)
    scale = jnp.broadcast_to(
        jnp.pad(sizes_f, ((0, 0), (0, 6)))[:, :, None], (b, 8, _LANES))

    body = functools.partial(_body, nrows, nblk, nq, c, k)
    scores, labels, boxsel = pl.pallas_call(
        body,
        grid=(b,),
        in_specs=[
            pl.BlockSpec((1, nrows, _LANES), lambda i: (i, 0, 0)),
            pl.BlockSpec((1, 8, nq), lambda i: (i, 0, 0)),
            pl.BlockSpec((1, 8, _LANES), lambda i: (i, 0, 0)),
        ],
        out_specs=[
            pl.BlockSpec((1, 8, _SLOTS), lambda i: (i, 0, 0)),
            pl.BlockSpec((1, 8, _SLOTS), lambda i: (i, 0, 0)),
            pl.BlockSpec((1, 8, _SLOTS), lambda i: (i, 0, 0)),
        ],
        out_shape=[
            jax.ShapeDtypeStruct((b, 8, _SLOTS), jnp.float32),
            jax.ShapeDtypeStruct((b, 8, _SLOTS), jnp.int32),
            jax.ShapeDtypeStruct((b, 8, _SLOTS), jnp.float32),
        ],
        scratch_shapes=[
            pltpu.VMEM((nrows, _LANES), jnp.int32),
            pltpu.VMEM((nrows, _LANES), jnp.int16),
            pltpu.VMEM((nrows, _LANES), jnp.int32),
            pltpu.VMEM((8, _LANES), jnp.int32),
        ],
        compiler_params=pltpu.CompilerParams(
            dimension_semantics=("parallel",)),
    )(flat, boxes_t, scale)

    top_scores = scores[:, 0, :k]
    out_labels = labels[:, 0, :k]
    out_boxes = jnp.transpose(boxsel[:, :4, :k], (0, 2, 1))
    return (out_labels, out_boxes, top_scores)


def kernel(pred_logits, pred_boxes, orig_target_sizes, threshold):
    return _run(pred_logits, pred_boxes, orig_target_sizes, _K)


# speedup vs baseline: 1.0867x; 1.0522x over previous
import functools

import jax
import jax.numpy as jnp
from jax.experimental import pallas as pl
from jax.experimental.pallas import tpu as pltpu

_B, _NQ, _C = 64, 5000, 80
_K = 300
_LANES = 128
_BLK_ROWS = 32
_SLOTS = 512
_BIG = 1 << 30


def _body(nrows, nblk, nq, c, k, logits_ref, boxes_ref, scale_ref,
          scores_out, labels_out, boxsel_out, si_ref, h16_ref, y_ref, bm_ref):
    s = jax.nn.sigmoid(logits_ref[0])
    si = jax.lax.bitcast_convert_type(s, jnp.int32)
    si_ref[...] = si
    h16_ref[...] = (si >> 16).astype(jnp.int16)

    def count16(mid):
        cmp = (h16_ref[...] > mid.astype(jnp.int16)).astype(jnp.int16)
        return jnp.sum(jnp.sum(cmp, axis=0).astype(jnp.int32))

    def bis_body(cntf, kk):
        def body(_, carry):
            lo, hi = carry
            mid = (lo + hi) // 2
            pred = cntf(mid) < kk
            return (jnp.where(pred, lo, mid), jnp.where(pred, mid, hi))
        return body

    _, t16 = jax.lax.fori_loop(
        0, 15, bis_body(count16, k), (jnp.int32(-1), jnp.int32(0x3F80)))
    m_hi = count16(t16)
    kb = k - m_hi

    low = jnp.where(si >> 16 == t16, (si & 0xFFFF) - 32768, -32768)
    h16_ref[...] = low.astype(jnp.int16)
    _, tlo = jax.lax.fori_loop(
        0, 17, bis_body(count16, kb), (jnp.int32(-32769), jnp.int32(32767)))
    tbits = (t16 << 16) + (tlo + 32768)

    m = jnp.sum((si_ref[...] > tbits).astype(jnp.int32))
    eq_total = jnp.sum((si_ref[...] == tbits).astype(jnp.int32))
    ties_needed = k - m

    fi = (jax.lax.broadcasted_iota(jnp.int32, (nrows, _LANES), 0) * _LANES
          + jax.lax.broadcasted_iota(jnp.int32, (nrows, _LANES), 1))

    def tie_bisect():
        def tie_body(_, carry):
            lo, hi = carry
            mid = (lo + hi) // 2
            eqfi = jnp.where(si_ref[...] == tbits, fi, jnp.int32(_BIG))
            cnt = jnp.sum((eqfi < mid).astype(jnp.int32))
            pred = cnt >= ties_needed
            return (jnp.where(pred, lo, mid), jnp.where(pred, mid, hi))
        return jax.lax.fori_loop(
            0, 20, tie_body, (jnp.int32(-1), jnp.int32(nrows * _LANES)))[1]

    istar = jax.lax.cond(eq_total == ties_needed,
                         lambda: jnp.int32(_BIG), tie_bisect)

    sel = (si_ref[...] > tbits) | ((si_ref[...] == tbits) & (fi < istar))
    y_ref[...] = jnp.where(sel, si_ref[...], jnp.int32(-1))

    yb = y_ref[...].reshape(nblk, _BLK_ROWS, _LANES)
    bm0 = jnp.max(yb, axis=(1, 2))
    bmrow = jnp.concatenate(
        [bm0, jnp.full((_LANES - nblk,), -1, jnp.int32)]).reshape(1, _LANES)

    lane_iota = jax.lax.broadcasted_iota(jnp.int32, (1, _LANES), 1)
    slot_iota = jax.lax.broadcasted_iota(jnp.int32, (1, _SLOTS), 1)
    fib = (jax.lax.broadcasted_iota(jnp.int32, (_BLK_ROWS, _LANES), 0) * _LANES
           + jax.lax.broadcasted_iota(jnp.int32, (_BLK_ROWS, _LANES), 1))

    def extract_body(cslot, carry):
        sv, fv, bmr = carry
        v = jnp.max(bmr, axis=1, keepdims=True)
        kstar = jnp.min(jnp.where(bmr == v, lane_iota, jnp.int32(_LANES)))
        blk = y_ref[pl.ds(kstar * _BLK_ROWS, _BLK_ROWS), :]
        p = jnp.min(jnp.where(blk == v, fib, jnp.int32(_BIG)),
                    keepdims=True).reshape(1, 1)
        oh = slot_iota == cslot
        sv = sv + jnp.where(oh, v, 0)
        fv = fv + jnp.where(oh, kstar * (_BLK_ROWS * _LANES) + p, 0)
        blk2 = jnp.where(fib == p, jnp.int32(-1), blk)
        y_ref[pl.ds(kstar * _BLK_ROWS, _BLK_ROWS), :] = blk2
        newmax = jnp.max(blk2, keepdims=True).reshape(1, 1)
        bmr = jnp.where(lane_iota == kstar, newmax, bmr)
        return (sv, fv, bmr)

    sv0 = jnp.zeros((1, _SLOTS), jnp.int32)
    sv, fv, _ = jax.lax.fori_loop(0, k, extract_body, (sv0, sv0, bmrow))

    scores_out[0, 0:1, :] = jax.lax.bitcast_convert_type(sv, jnp.float32)
    q = fv // c
    labels_out[0, 0:1, :] = fv - q * c

    br = boxes_ref[0]
    w_sc = scale_ref[0, 0, 0]
    h_sc = scale_ref[0, 1, 0]
    cx, cy, bw, bh = br[0:1], br[1:2], br[2:3], br[3:4]
    x1 = (cx - 0.5 * bw) * w_sc
    y1 = (cy - 0.5 * bh) * h_sc
    x2 = (cx + 0.5 * bw) * w_sc
    y2 = (cy + 0.5 * bh) * h_sc
    xyxy = jnp.concatenate([x1, y1, x2, y2, jnp.zeros((4, nq), jnp.float32)])
    q_iota = jax.lax.broadcasted_iota(jnp.int32, (nq, _SLOTS), 0)
    oht = (q_iota == q).astype(jnp.float32)
    boxsel_out[0] = jax.lax.dot_general(
        xyxy, oht, (((1,), (0,)), ((), ())),
        preferred_element_type=jnp.float32)


def _run(pred_logits, pred_boxes, orig_target_sizes, k):
    b, nq, c = pred_logits.shape
    flat = pred_logits.reshape(b, (nq * c) // _LANES, _LANES)
    nrows = -(-flat.shape[1] // _BLK_ROWS) * _BLK_ROWS
    nblk = nrows // _BLK_ROWS
    flat = jnp.pad(flat, ((0, 0), (0, nrows - flat.shape[1]), (0, 0)),
                   constant_values=-1e30)
    boxes_t = jnp.pad(jnp.transpose(pred_boxes, (0, 2, 1)),
                      ((0, 0), (0, 4), (0, 0)))
    sizes_f = orig_target_sizes.astype(jnp.float32)
    scale = jnp.broadcast_to(
        jnp.pad(sizes_f, ((0, 0), (0, 6)))[:, :, None], (b, 8, _LANES))

    body = functools.partial(_body, nrows, nblk, nq, c, k)
    scores, labels, boxsel = pl.pallas_call(
        body,
        grid=(b,),
        in_specs=[
            pl.BlockSpec((1, nrows, _LANES), lambda i: (i, 0, 0)),
            pl.BlockSpec((1, 8, nq), lambda i: (i, 0, 0)),
            pl.BlockSpec((1, 8, _LANES), lambda i: (i, 0, 0)),
        ],
        out_specs=[
            pl.BlockSpec((1, 8, _SLOTS), lambda i: (i, 0, 0)),
            pl.BlockSpec((1, 8, _SLOTS), lambda i: (i, 0, 0)),
            pl.BlockSpec((1, 8, _SLOTS), lambda i: (i, 0, 0)),
        ],
        out_shape=[
            jax.ShapeDtypeStruct((b, 8, _SLOTS), jnp.float32),
            jax.ShapeDtypeStruct((b, 8, _SLOTS), jnp.int32),
            jax.ShapeDtypeStruct((b, 8, _SLOTS), jnp.float32),
        ],
        scratch_shapes=[
            pltpu.VMEM((nrows, _LANES), jnp.int32),
            pltpu.VMEM((nrows, _LANES), jnp.int16),
            pltpu.VMEM((nrows, _LANES), jnp.int32),
            pltpu.VMEM((8, _LANES), jnp.int32),
        ],
        compiler_params=pltpu.CompilerParams(
            dimension_semantics=("parallel",)),
    )(flat, boxes_t, scale)

    top_scores = scores[:, 0, :k]
    out_labels = labels[:, 0, :k]
    out_boxes = jnp.transpose(boxsel[:, :4, :k], (0, 2, 1))
    return (out_labels, out_boxes, top_scores)


def kernel(pred_logits, pred_boxes, orig_target_sizes, threshold):
    return _run(pred_logits, pred_boxes, orig_target_sizes, _K)
